# baseline (speedup 1.0000x reference)
import jax
from jax.experimental import pallas as pl
from jax.experimental.pallas import tpu as pltpu


def _copy_body(emb_in, masks_in, emb_out, masks_out, sem_e, sem_m):
    ce = pltpu.make_async_copy(emb_in, emb_out, sem_e)
    cm = pltpu.make_async_copy(masks_in, masks_out, sem_m)
    ce.start()
    cm.start()
    ce.wait()
    cm.wait()


def kernel(embeddings, blade_masks):
    emb_out, masks_out = pl.pallas_call(
        _copy_body,
        in_specs=[
            pl.BlockSpec(memory_space=pl.ANY),
            pl.BlockSpec(memory_space=pl.ANY),
        ],
        out_specs=[
            pl.BlockSpec(memory_space=pl.ANY),
            pl.BlockSpec(memory_space=pl.ANY),
        ],
        out_shape=[
            jax.ShapeDtypeStruct(embeddings.shape, embeddings.dtype),
            jax.ShapeDtypeStruct(blade_masks.shape, blade_masks.dtype),
        ],
        scratch_shapes=[pltpu.SemaphoreType.DMA, pltpu.SemaphoreType.DMA],
    )(embeddings, blade_masks)
    return (emb_out, masks_out)
